# baseline (speedup 1.0000x reference)
import functools

import jax
import jax.numpy as jnp
from jax import lax
from jax.experimental import pallas as pl
from jax.experimental.pallas import tpu as pltpu
from jax.experimental.pallas import tpu_sc as plsc

N = 10000
E = 320000
D = 128
V = 100000
G = 16
C = 8

NP = 10240
NW = 32
EW = 10112
E_PAD = NW * EW
CH = 128
GCH = 80
RPS = NP // 16

_HIGH = jax.lax.Precision.HIGHEST


def _mesh():
    return plsc.VectorSubcoreMesh(core_axis_name="c", subcore_axis_name="s")


GCH = 80
NEC = (NP // NW) // GCH


def _emb_deg_pass(emb, featp, dst3, za, ones_e):
    @functools.partial(
        pl.kernel,
        out_type=(jax.ShapeDtypeStruct((NP, D), jnp.float32),
                  jax.ShapeDtypeStruct((2, NP, D), jnp.float32)),
        mesh=_mesh(),
        scratch_types=[
            pltpu.VMEM((NCH, CH), jnp.int32),
            pltpu.VMEM((CH, D), jnp.float32),
            pltpu.VMEM((GCH,), jnp.int32),
            pltpu.VMEM((GCH, D), jnp.float32),
            pltpu.VMEM_SHARED((NP, D), jnp.float32),
            pltpu.SemaphoreType.DMA,
            pltpu.SemaphoreType.DMA,
        ],
    )
    def k(emb_hbm, feat_hbm, dst_hbm, za_hbm, ones_hbm,
          h0_out, deg_out, dst_v, ones_v, eidx, erows, deg_sh, sem, sege):
        c = lax.axis_index("c")
        s = lax.axis_index("s")
        wid = s * 2 + c

        pltpu.sync_copy(za_hbm.at[pl.ds(s * RPS, RPS)],
                        deg_sh.at[pl.ds(s * RPS, RPS)])
        pltpu.sync_copy(ones_hbm, ones_v)
        pltpu.sync_copy(dst_hbm.at[wid], dst_v)
        plsc.subcore_barrier()

        def one_wait():
            pltpu.make_async_copy(ones_v, deg_sh.at[dst_v.at[0]], sem).wait()

        def emb_chunk(kk):
            base = wid * (NP // NW) + kk * GCH
            pltpu.sync_copy(feat_hbm.at[pl.ds(base, GCH)], eidx)
            pltpu.async_copy(emb_hbm.at[eidx], erows, sege).wait()
            pltpu.sync_copy(erows, h0_out.at[pl.ds(base, GCH)])

        seg = NCH // NEC
        for kk in range(NEC):
            emb_chunk(kk)
            hi = (kk + 1) * seg if kk + 1 < NEC else NCH

            @pl.loop(kk * seg, hi, step=1)
            def _(j):
                pltpu.async_copy(ones_v, deg_sh.at[dst_v.at[j]], sem,
                                 add=True)

                @pl.when(j >= 8)
                def _():
                    one_wait()

        @pl.loop(0, 8, step=1)
        def _(j):
            one_wait()

        plsc.subcore_barrier()
        pltpu.sync_copy(deg_sh.at[pl.ds(s * RPS, RPS)],
                        deg_out.at[c].at[pl.ds(s * RPS, RPS)])

    return k(emb, featp, dst3, za, ones_e)


NCH = EW // CH


def _agg_pass(h, src3, dst3, za):
    @functools.partial(
        pl.kernel,
        out_type=jax.ShapeDtypeStruct((2, NP, D), jnp.float32),
        mesh=_mesh(),
        scratch_types=[
            pltpu.VMEM((NCH, CH), jnp.int32),
            pltpu.VMEM((CH,), jnp.int32),
            pltpu.VMEM((CH,), jnp.int32),
            pltpu.VMEM((CH, D), jnp.float32),
            pltpu.VMEM((CH, D), jnp.float32),
            pltpu.VMEM_SHARED((NP, D), jnp.float32),
            pltpu.SemaphoreType.DMA,
            pltpu.SemaphoreType.DMA,
            pltpu.SemaphoreType.DMA,
            pltpu.SemaphoreType.DMA,
        ],
    )
    def k(h_hbm, src_hbm, dst_hbm, za_hbm, agg_out,
          dst_v, srcb0, srcb1, rows_a, rows_b, agg_sh, sa, sb, sl0, sl1):
        c = lax.axis_index("c")
        s = lax.axis_index("s")
        wid = s * 2 + c

        def ld_start(j, sbuf, sem):
            pltpu.async_copy(src_hbm.at[wid].at[j], sbuf, sem)

        def ld_wait(j, sbuf, sem):
            pltpu.make_async_copy(src_hbm.at[wid].at[j], sbuf, sem).wait()

        def g_start(sbuf, buf, sem):
            pltpu.async_copy(h_hbm.at[sbuf], buf, sem)

        def g_wait(sbuf, buf, sem):
            pltpu.make_async_copy(h_hbm.at[sbuf], buf, sem).wait()

        def s_start(buf, j, sem):
            pltpu.async_copy(buf, agg_sh.at[dst_v.at[j]], sem, add=True)

        def s_wait(buf, j, sem):
            pltpu.make_async_copy(buf, agg_sh.at[dst_v.at[j]], sem).wait()

        pltpu.sync_copy(src_hbm.at[wid].at[0], srcb0)
        g_start(srcb0, rows_a, sa)
        ld_start(1, srcb1, sl1)

        pltpu.sync_copy(za_hbm.at[pl.ds(s * RPS, RPS)],
                        agg_sh.at[pl.ds(s * RPS, RPS)])
        pltpu.sync_copy(dst_hbm.at[wid], dst_v)
        plsc.subcore_barrier()

        g_wait(srcb0, rows_a, sa)
        s_start(rows_a, 0, sa)
        ld_wait(1, srcb1, sl1)
        g_start(srcb1, rows_b, sb)
        ld_start(2, srcb0, sl0)
        g_wait(srcb1, rows_b, sb)
        s_start(rows_b, 1, sb)
        ld_wait(2, srcb0, sl0)
        s_wait(rows_a, 0, sa)
        g_start(srcb0, rows_a, sa)
        ld_start(3, srcb1, sl1)

        @pl.loop(2, NCH - 1, step=2)
        def _(j):
            ld_wait(j + 1, srcb1, sl1)
            s_wait(rows_b, j - 1, sb)
            g_start(srcb1, rows_b, sb)
            g_wait(srcb0, rows_a, sa)
            s_start(rows_a, j, sa)
            ld_start(j + 2, srcb0, sl0)
            ld_wait(j + 2, srcb0, sl0)
            s_wait(rows_a, j, sa)
            g_start(srcb0, rows_a, sa)
            g_wait(srcb1, rows_b, sb)
            s_start(rows_b, j + 1, sb)
            ld_start(j + 3, srcb1, sl1)

        g_wait(srcb0, rows_a, sa)
        s_start(rows_a, NCH - 1, sa)
        ld_wait(NCH, srcb1, sl1)
        s_wait(rows_b, NCH - 2, sb)
        s_wait(rows_a, NCH - 1, sa)

        plsc.subcore_barrier()
        pltpu.sync_copy(agg_sh.at[pl.ds(s * RPS, RPS)],
                        agg_out.at[c].at[pl.ds(s * RPS, RPS)])

    return k(h, src3, dst3, za)


BLK = 1024


def _dense_body(h_ref, agg_ref, deg_ref, ws_ref, wn_ref, b_ref, out_ref):
    a = agg_ref[0] + agg_ref[1]
    d = deg_ref[0, :, 0:1] + deg_ref[1, :, 0:1]
    hn = a / jnp.maximum(d, 1.0)
    z = (jnp.dot(h_ref[...], ws_ref[...], precision=_HIGH)
         + jnp.dot(hn, wn_ref[...], precision=_HIGH)
         + b_ref[...])
    out_ref[...] = jnp.maximum(z, 0.0)


def _dense(h, agg, deg, ws, wn, b):
    return pl.pallas_call(
        _dense_body,
        grid=(NP // BLK,),
        in_specs=[
            pl.BlockSpec((BLK, D), lambda i: (i, 0)),
            pl.BlockSpec((2, BLK, D), lambda i: (0, i, 0)),
            pl.BlockSpec((2, BLK, D), lambda i: (0, i, 0)),
            pl.BlockSpec((D, D), lambda i: (0, 0)),
            pl.BlockSpec((D, D), lambda i: (0, 0)),
            pl.BlockSpec((1, D), lambda i: (0, 0)),
        ],
        out_specs=pl.BlockSpec((BLK, D), lambda i: (i, 0)),
        out_shape=jax.ShapeDtypeStruct((NP, D), jnp.float32),
    )(h, agg, deg, ws, wn, b)


def _readout_body(h_ref, gid_ref, e1w, e1b, e2w, e2b, e3w, e3b, e4w, e4b,
                  pw, pb, logp_ref, hid_ref):
    gid = gid_ref[...]
    gi = jax.lax.broadcasted_iota(jnp.int32, (NP, G), 1)
    oh = (gid == gi).astype(jnp.float32)
    h = h_ref[...]
    dn = (((0,), (0,)), ((), ()))
    hg_sum = jax.lax.dot_general(oh, h, dn, precision=_HIGH)
    gram = jax.lax.dot_general(oh, oh, dn, precision=_HIGH)
    eye = (jax.lax.broadcasted_iota(jnp.int32, (G, G), 0)
           == jax.lax.broadcasted_iota(jnp.int32, (G, G), 1)
           ).astype(jnp.float32)
    cnt = jnp.sum(gram * eye, axis=1, keepdims=True)
    hg = hg_sum / jnp.maximum(cnt, 1.0)
    hid = jnp.maximum(jnp.dot(hg, e1w[...], precision=_HIGH) + e1b[...], 0.0)
    h2 = jnp.maximum(jnp.dot(hid, e2w[...], precision=_HIGH) + e2b[...], 0.0)
    h3 = jnp.maximum(jnp.dot(h2, e3w[...], precision=_HIGH) + e3b[...], 0.0)
    h4 = jnp.maximum(jnp.dot(h3, e4w[...], precision=_HIGH) + e4b[...], 0.0)
    y = jnp.dot(h4, pw[...], precision=_HIGH) + pb[...]
    m = jnp.max(y, axis=1, keepdims=True)
    z = y - m
    lse = jnp.log(jnp.sum(jnp.exp(z), axis=1, keepdims=True))
    logp_ref[...] = z - lse
    hid_ref[...] = hid


def _readout(h2, gidp, e1w, e1b, e2w, e2b, e3w, e3b, e4w, e4b, pw, pb):
    return pl.pallas_call(
        _readout_body,
        out_shape=(jax.ShapeDtypeStruct((G, C), jnp.float32),
                   jax.ShapeDtypeStruct((G, D), jnp.float32)),
    )(h2, gidp, e1w, e1b, e2w, e2b, e3w, e3b, e4w, e4b, pw, pb)


def kernel(feat, edge_index, graph_ids, emb,
           W_self0, b_self0, W_neigh0, b_neigh0,
           W_self1, b_self1, W_neigh1, b_neigh1,
           e1_W, e1_b, e2_W, e2_b, e3_W, e3_b, e4_W, e4_b,
           pred_W, pred_b):
    src = edge_index[0]
    dst = edge_index[1]
    pe = E_PAD - E
    srcp = jnp.concatenate(
        [src, (jnp.arange(pe, dtype=jnp.int32) * 37) % N])
    dstp = jnp.concatenate(
        [dst, N + (jnp.arange(pe, dtype=jnp.int32) % 16)])
    featp = jnp.concatenate([feat, jnp.zeros((NP - N,), jnp.int32)])
    gidp = jnp.concatenate(
        [graph_ids, jnp.full((NP - N,), G, jnp.int32)]).reshape(NP, 1)
    za = jnp.zeros((NP, D), jnp.float32)
    ones_e = jnp.ones((CH, D), jnp.float32)
    b0 = (b_self0 + b_neigh0).reshape(1, D)
    b1 = (b_self1 + b_neigh1).reshape(1, D)

    src3 = jnp.concatenate(
        [srcp.reshape(NW, NCH, CH), jnp.zeros((NW, 1, CH), jnp.int32)], axis=1)
    dst3 = dstp.reshape(NW, NCH, CH)
    h0, deg2 = _emb_deg_pass(emb, featp, dst3, za, ones_e)
    agg0 = _agg_pass(h0, src3, dst3, za)
    h1 = _dense(h0, agg0, deg2, W_self0, W_neigh0, b0)
    agg1 = _agg_pass(h1, src3, dst3, za)
    h2 = _dense(h1, agg1, deg2, W_self1, W_neigh1, b1)
    logp, hidden = _readout(h2, gidp,
                            e1_W, e1_b.reshape(1, D), e2_W, e2_b.reshape(1, D),
                            e3_W, e3_b.reshape(1, D), e4_W, e4_b.reshape(1, D),
                            pred_W, pred_b.reshape(1, C))
    return (logp, hidden)


# speedup vs baseline: 1.0128x; 1.0128x over previous
import functools

import jax
import jax.numpy as jnp
from jax import lax
from jax.experimental import pallas as pl
from jax.experimental.pallas import tpu as pltpu
from jax.experimental.pallas import tpu_sc as plsc

N = 10000
E = 320000
D = 128
V = 100000
G = 16
C = 8

NP = 10240
NW = 32
EW = 10112
E_PAD = NW * EW
CH = 128
GCH = 80
RPS = NP // 16

_HIGH = jax.lax.Precision.HIGHEST


def _mesh():
    return plsc.VectorSubcoreMesh(core_axis_name="c", subcore_axis_name="s")


GCH = 80
NEC = (NP // NW) // GCH


def _emb_deg_pass(emb, featp, dst3, za, ones_e):
    @functools.partial(
        pl.kernel,
        out_type=(jax.ShapeDtypeStruct((NP, D), jnp.float32),
                  jax.ShapeDtypeStruct((2, NP, D), jnp.float32)),
        mesh=_mesh(),
        scratch_types=[
            pltpu.VMEM((NCH, CH), jnp.int32),
            pltpu.VMEM((CH, D), jnp.float32),
            pltpu.VMEM((GCH,), jnp.int32),
            pltpu.VMEM((GCH, D), jnp.float32),
            pltpu.VMEM_SHARED((NP, D), jnp.float32),
            pltpu.SemaphoreType.DMA,
            pltpu.SemaphoreType.DMA,
        ],
    )
    def k(emb_hbm, feat_hbm, dst_hbm, za_hbm, ones_hbm,
          h0_out, deg_out, dst_v, ones_v, eidx, erows, deg_sh, sem, sege):
        c = lax.axis_index("c")
        s = lax.axis_index("s")
        wid = s * 2 + c

        pltpu.sync_copy(za_hbm.at[pl.ds(s * RPS, RPS)],
                        deg_sh.at[pl.ds(s * RPS, RPS)])
        pltpu.sync_copy(ones_hbm, ones_v)
        pltpu.sync_copy(dst_hbm.at[wid], dst_v)
        plsc.subcore_barrier()

        def one_wait():
            pltpu.make_async_copy(ones_v, deg_sh.at[dst_v.at[0]], sem).wait()

        def emb_chunk(kk):
            base = wid * (NP // NW) + kk * GCH
            pltpu.sync_copy(feat_hbm.at[pl.ds(base, GCH)], eidx)
            pltpu.async_copy(emb_hbm.at[eidx], erows, sege).wait()
            pltpu.sync_copy(erows, h0_out.at[pl.ds(base, GCH)])

        seg = NCH // NEC
        for kk in range(NEC):
            emb_chunk(kk)
            hi = (kk + 1) * seg if kk + 1 < NEC else NCH

            @pl.loop(kk * seg, hi, step=1)
            def _(j):
                pltpu.async_copy(ones_v, deg_sh.at[dst_v.at[j]], sem,
                                 add=True)

                @pl.when(j >= 8)
                def _():
                    one_wait()

        @pl.loop(0, 8, step=1)
        def _(j):
            one_wait()

        plsc.subcore_barrier()
        pltpu.sync_copy(deg_sh.at[pl.ds(s * RPS, RPS)],
                        deg_out.at[c].at[pl.ds(s * RPS, RPS)])

    return k(emb, featp, dst3, za, ones_e)


NCH = EW // CH


def _agg_pass(h, src3, dst3, za):
    @functools.partial(
        pl.kernel,
        out_type=jax.ShapeDtypeStruct((2, NP, D), jnp.float32),
        mesh=_mesh(),
        scratch_types=[
            pltpu.VMEM((NCH, CH), jnp.int32),
            pltpu.VMEM((CH,), jnp.int32),
            pltpu.VMEM((CH,), jnp.int32),
            pltpu.VMEM((CH, D), jnp.float32),
            pltpu.VMEM((CH, D), jnp.float32),
            pltpu.VMEM_SHARED((NP, D), jnp.float32),
            pltpu.SemaphoreType.DMA,
            pltpu.SemaphoreType.DMA,
            pltpu.SemaphoreType.DMA,
            pltpu.SemaphoreType.DMA,
        ],
    )
    def k(h_hbm, src_hbm, dst_hbm, za_hbm, agg_out,
          dst_v, srcb0, srcb1, rows_a, rows_b, agg_sh, sa, sb, sl0, sl1):
        c = lax.axis_index("c")
        s = lax.axis_index("s")
        wid = s * 2 + c

        def ld_start(j, sbuf, sem):
            pltpu.async_copy(src_hbm.at[wid].at[j], sbuf, sem)

        def ld_wait(j, sbuf, sem):
            pltpu.make_async_copy(src_hbm.at[wid].at[j], sbuf, sem).wait()

        def g_start(sbuf, buf, sem):
            pltpu.async_copy(h_hbm.at[sbuf], buf, sem)

        def g_wait(sbuf, buf, sem):
            pltpu.make_async_copy(h_hbm.at[sbuf], buf, sem).wait()

        def s_start(buf, j, sem):
            pltpu.async_copy(buf, agg_sh.at[dst_v.at[j]], sem, add=True)

        def s_wait(buf, j, sem):
            pltpu.make_async_copy(buf, agg_sh.at[dst_v.at[j]], sem).wait()

        pltpu.sync_copy(src_hbm.at[wid].at[0], srcb0)
        g_start(srcb0, rows_a, sa)
        ld_start(1, srcb1, sl1)

        pltpu.sync_copy(za_hbm.at[pl.ds(s * RPS, RPS)],
                        agg_sh.at[pl.ds(s * RPS, RPS)])
        pltpu.sync_copy(dst_hbm.at[wid], dst_v)
        plsc.subcore_barrier()

        g_wait(srcb0, rows_a, sa)
        s_start(rows_a, 0, sa)
        ld_wait(1, srcb1, sl1)
        g_start(srcb1, rows_b, sb)
        ld_start(2, srcb0, sl0)
        g_wait(srcb1, rows_b, sb)
        s_start(rows_b, 1, sb)
        ld_wait(2, srcb0, sl0)
        s_wait(rows_a, 0, sa)
        g_start(srcb0, rows_a, sa)
        ld_start(3, srcb1, sl1)

        @pl.loop(2, NCH - 1, step=2)
        def _(j):
            ld_wait(j + 1, srcb1, sl1)
            s_wait(rows_b, j - 1, sb)
            g_start(srcb1, rows_b, sb)
            g_wait(srcb0, rows_a, sa)
            s_start(rows_a, j, sa)
            ld_start(j + 2, srcb0, sl0)
            ld_wait(j + 2, srcb0, sl0)
            s_wait(rows_a, j, sa)
            g_start(srcb0, rows_a, sa)
            g_wait(srcb1, rows_b, sb)
            s_start(rows_b, j + 1, sb)
            ld_start(j + 3, srcb1, sl1)

        g_wait(srcb0, rows_a, sa)
        s_start(rows_a, NCH - 1, sa)
        ld_wait(NCH, srcb1, sl1)
        s_wait(rows_b, NCH - 2, sb)
        s_wait(rows_a, NCH - 1, sa)

        plsc.subcore_barrier()
        pltpu.sync_copy(agg_sh.at[pl.ds(s * RPS, RPS)],
                        agg_out.at[c].at[pl.ds(s * RPS, RPS)])

    return k(h, src3, dst3, za)


BLK = 2560


def _dense_body(h_ref, agg_ref, deg_ref, ws_ref, wn_ref, b_ref, out_ref):
    a = agg_ref[0] + agg_ref[1]
    d = deg_ref[0, :, 0:1] + deg_ref[1, :, 0:1]
    hn = a / jnp.maximum(d, 1.0)
    z = (jnp.dot(h_ref[...], ws_ref[...], precision=_HIGH)
         + jnp.dot(hn, wn_ref[...], precision=_HIGH)
         + b_ref[...])
    out_ref[...] = jnp.maximum(z, 0.0)


def _dense(h, agg, deg, ws, wn, b):
    return pl.pallas_call(
        _dense_body,
        grid=(NP // BLK,),
        in_specs=[
            pl.BlockSpec((BLK, D), lambda i: (i, 0)),
            pl.BlockSpec((2, BLK, D), lambda i: (0, i, 0)),
            pl.BlockSpec((2, BLK, D), lambda i: (0, i, 0)),
            pl.BlockSpec((D, D), lambda i: (0, 0)),
            pl.BlockSpec((D, D), lambda i: (0, 0)),
            pl.BlockSpec((1, D), lambda i: (0, 0)),
        ],
        out_specs=pl.BlockSpec((BLK, D), lambda i: (i, 0)),
        out_shape=jax.ShapeDtypeStruct((NP, D), jnp.float32),
    )(h, agg, deg, ws, wn, b)


def _readout_body(h_ref, gid_ref, e1w, e1b, e2w, e2b, e3w, e3b, e4w, e4b,
                  pw, pb, logp_ref, hid_ref):
    gid = gid_ref[...]
    gi = jax.lax.broadcasted_iota(jnp.int32, (NP, G), 1)
    oh = (gid == gi).astype(jnp.float32)
    h = h_ref[...]
    dn = (((0,), (0,)), ((), ()))
    hg_sum = jax.lax.dot_general(oh, h, dn, precision=_HIGH)
    gram = jax.lax.dot_general(oh, oh, dn, precision=_HIGH)
    eye = (jax.lax.broadcasted_iota(jnp.int32, (G, G), 0)
           == jax.lax.broadcasted_iota(jnp.int32, (G, G), 1)
           ).astype(jnp.float32)
    cnt = jnp.sum(gram * eye, axis=1, keepdims=True)
    hg = hg_sum / jnp.maximum(cnt, 1.0)
    hid = jnp.maximum(jnp.dot(hg, e1w[...], precision=_HIGH) + e1b[...], 0.0)
    h2 = jnp.maximum(jnp.dot(hid, e2w[...], precision=_HIGH) + e2b[...], 0.0)
    h3 = jnp.maximum(jnp.dot(h2, e3w[...], precision=_HIGH) + e3b[...], 0.0)
    h4 = jnp.maximum(jnp.dot(h3, e4w[...], precision=_HIGH) + e4b[...], 0.0)
    y = jnp.dot(h4, pw[...], precision=_HIGH) + pb[...]
    m = jnp.max(y, axis=1, keepdims=True)
    z = y - m
    lse = jnp.log(jnp.sum(jnp.exp(z), axis=1, keepdims=True))
    logp_ref[...] = z - lse
    hid_ref[...] = hid


def _readout(h2, gidp, e1w, e1b, e2w, e2b, e3w, e3b, e4w, e4b, pw, pb):
    return pl.pallas_call(
        _readout_body,
        out_shape=(jax.ShapeDtypeStruct((G, C), jnp.float32),
                   jax.ShapeDtypeStruct((G, D), jnp.float32)),
    )(h2, gidp, e1w, e1b, e2w, e2b, e3w, e3b, e4w, e4b, pw, pb)


def kernel(feat, edge_index, graph_ids, emb,
           W_self0, b_self0, W_neigh0, b_neigh0,
           W_self1, b_self1, W_neigh1, b_neigh1,
           e1_W, e1_b, e2_W, e2_b, e3_W, e3_b, e4_W, e4_b,
           pred_W, pred_b):
    src = edge_index[0]
    dst = edge_index[1]
    pe = E_PAD - E
    srcp = jnp.concatenate(
        [src, (jnp.arange(pe, dtype=jnp.int32) * 37) % N])
    dstp = jnp.concatenate(
        [dst, N + (jnp.arange(pe, dtype=jnp.int32) % 16)])
    featp = jnp.concatenate([feat, jnp.zeros((NP - N,), jnp.int32)])
    gidp = jnp.concatenate(
        [graph_ids, jnp.full((NP - N,), G, jnp.int32)]).reshape(NP, 1)
    za = jnp.zeros((NP, D), jnp.float32)
    ones_e = jnp.ones((CH, D), jnp.float32)
    b0 = (b_self0 + b_neigh0).reshape(1, D)
    b1 = (b_self1 + b_neigh1).reshape(1, D)

    src3 = jnp.concatenate(
        [srcp.reshape(NW, NCH, CH), jnp.zeros((NW, 1, CH), jnp.int32)], axis=1)
    dst3 = dstp.reshape(NW, NCH, CH)
    h0, deg2 = _emb_deg_pass(emb, featp, dst3, za, ones_e)
    agg0 = _agg_pass(h0, src3, dst3, za)
    h1 = _dense(h0, agg0, deg2, W_self0, W_neigh0, b0)
    agg1 = _agg_pass(h1, src3, dst3, za)
    h2 = _dense(h1, agg1, deg2, W_self1, W_neigh1, b1)
    logp, hidden = _readout(h2, gidp,
                            e1_W, e1_b.reshape(1, D), e2_W, e2_b.reshape(1, D),
                            e3_W, e3_b.reshape(1, D), e4_W, e4_b.reshape(1, D),
                            pred_W, pred_b.reshape(1, C))
    return (logp, hidden)
